# Initial kernel scaffold; baseline (speedup 1.0000x reference)
import functools

import jax
import jax.numpy as jnp
from jax import lax
from jax.experimental import pallas as pl
from jax.experimental.pallas import tpu as pltpu
from jax.experimental.pallas import tpu_sc as plsc

NUM_FIELDS = 26
VOCAB = 100000
EMBED_DIM = 32
BATCH = 16384

_info = plsc.get_sparse_core_info()
NC, NS, L = _info.num_cores, _info.num_subcores, _info.num_lanes
NW = NC * NS
ROWS = BATCH * NUM_FIELDS
RPW = ROWS // NW
CHUNK = 128
NCHUNK = RPW // CHUNK
VPC = CHUNK // L


def _body(x_hbm, off_hbm, tab_hbm, out_hbm, idx_v, off_v, row_v, sem):
    wid = lax.axis_index("s") * NC + lax.axis_index("c")
    pltpu.sync_copy(x_hbm.at[wid], idx_v)
    pltpu.sync_copy(off_hbm, off_v)

    def do_chunk(j, _):
        for k in range(VPC):
            sl = pl.ds(k * L, L)
            idx_v[j, sl] = idx_v[j, sl] + off_v[j, sl]
        pltpu.async_copy(tab_hbm.at[idx_v.at[j]], row_v, sem).wait()
        pltpu.sync_copy(row_v, out_hbm.at[wid, pl.ds(j * CHUNK, CHUNK)])
        return 0

    lax.fori_loop(0, NCHUNK, do_chunk, 0)


@jax.jit
def kernel(x, tables):
    tab_flat = tables.reshape(NUM_FIELDS * VOCAB, EMBED_DIM)
    x_resh = x.reshape(NW, NCHUNK, CHUNK)
    offs = jnp.tile(jnp.arange(NUM_FIELDS, dtype=jnp.int32) * VOCAB,
                    RPW // NUM_FIELDS).reshape(NCHUNK, CHUNK)

    fn = pl.kernel(
        _body,
        out_type=jax.ShapeDtypeStruct((NW, RPW, EMBED_DIM), jnp.float32),
        mesh=plsc.VectorSubcoreMesh(core_axis_name="c", subcore_axis_name="s"),
        scratch_types=[
            pltpu.VMEM((NCHUNK, CHUNK), jnp.int32),
            pltpu.VMEM((NCHUNK, CHUNK), jnp.int32),
            pltpu.VMEM((CHUNK, EMBED_DIM), jnp.float32),
            pltpu.SemaphoreType.DMA,
        ],
    )
    out = fn(x_resh, offs, tab_flat)
    return out.reshape(BATCH, 1, NUM_FIELDS * EMBED_DIM)


# speedup vs baseline: 1.1480x; 1.1480x over previous
import functools

import jax
import jax.numpy as jnp
from jax import lax
from jax.experimental import pallas as pl
from jax.experimental.pallas import tpu as pltpu
from jax.experimental.pallas import tpu_sc as plsc

NUM_FIELDS = 26
VOCAB = 100000
EMBED_DIM = 32
BATCH = 16384

_info = plsc.get_sparse_core_info()
NC, NS, L = _info.num_cores, _info.num_subcores, _info.num_lanes
NW = NC * NS
ROWS = BATCH * NUM_FIELDS
RPW = ROWS // NW
CHUNK = 128
NCHUNK = RPW // CHUNK
VPC = CHUNK // L


def _body(x_hbm, off_hbm, tab_hbm, out_hbm, idx_v, off_v, row_v, sem):
    wid = lax.axis_index("s") * NC + lax.axis_index("c")
    pltpu.sync_copy(x_hbm.at[wid], idx_v)
    pltpu.sync_copy(off_hbm, off_v)

    def do_chunk(j, _):
        for k in range(VPC):
            sl = pl.ds(k * L, L)
            idx_v[j, sl] = idx_v[j, sl] + off_v[j, sl]
        pltpu.async_copy(tab_hbm.at[idx_v.at[j]], row_v, sem).wait()
        pltpu.sync_copy(row_v, out_hbm.at[wid, pl.ds(j * CHUNK, CHUNK)])
        return 0

    lax.fori_loop(0, NCHUNK, do_chunk, 0)


@jax.jit
def kernel(x, tables):
    tab_flat = tables.reshape(NUM_FIELDS * VOCAB, EMBED_DIM)
    x_resh = x.reshape(NW, NCHUNK, CHUNK)
    offs = jnp.tile(jnp.arange(NUM_FIELDS, dtype=jnp.int32) * VOCAB,
                    RPW // NUM_FIELDS).reshape(NCHUNK, CHUNK)

    fn = pl.kernel(
        _body,
        out_type=jax.ShapeDtypeStruct((NW, RPW, EMBED_DIM), jnp.float32),
        mesh=plsc.VectorSubcoreMesh(core_axis_name="c", subcore_axis_name="s"),
        scratch_types=[
            pltpu.VMEM((NCHUNK, CHUNK), jnp.int32),
            pltpu.VMEM((NCHUNK, CHUNK), jnp.int32),
            pltpu.VMEM((CHUNK, EMBED_DIM), jnp.float32),
            pltpu.SemaphoreType.DMA,
        ],
        compiler_params=pltpu.CompilerParams(use_tc_tiling_on_sc=False),
    )
    out = fn(x_resh, offs, tab_flat)
    return out.reshape(BATCH, 1, NUM_FIELDS * EMBED_DIM)


# speedup vs baseline: 1.2123x; 1.0560x over previous
import functools

import jax
import jax.numpy as jnp
from jax import lax
from jax.experimental import pallas as pl
from jax.experimental.pallas import tpu as pltpu
from jax.experimental.pallas import tpu_sc as plsc

NUM_FIELDS = 26
VOCAB = 100000
EMBED_DIM = 32
BATCH = 16384

_info = plsc.get_sparse_core_info()
NC, NS, L = _info.num_cores, _info.num_subcores, _info.num_lanes
NW = NC * NS
ROWS = BATCH * NUM_FIELDS
RPW = ROWS // NW
CHUNK = 832
NCHUNK = RPW // CHUNK
VPC = CHUNK // L
NSLOT = 3


def _body(x_hbm, off_hbm, tab_hbm, out_hbm, idx_v, off_v, bufs, sems_g, sems_w):
    wid = lax.axis_index("s") * NC + lax.axis_index("c")
    pltpu.sync_copy(x_hbm.at[wid], idx_v)
    pltpu.sync_copy(off_hbm, off_v)

    def prep_chunk(j, _):
        for k in range(VPC):
            sl = pl.ds(k * L, L)
            idx_v[j, sl] = idx_v[j, sl] + off_v[j, sl]
        return 0

    lax.fori_loop(0, NCHUNK, prep_chunk, 0)

    def gather(j, s):
        return pltpu.make_async_copy(tab_hbm.at[idx_v.at[j]], bufs[s], sems_g[s])

    def writeback(j, s):
        return pltpu.make_async_copy(
            bufs[s], out_hbm.at[wid, pl.ds(j * CHUNK, CHUNK)], sems_w[s])

    for j in range(NSLOT - 1):
        gather(j, j % NSLOT).start()
    for j in range(NCHUNK):
        s = j % NSLOT
        jn = j + NSLOT - 1
        if jn < NCHUNK:
            sn = jn % NSLOT
            if jn >= NSLOT:
                writeback(jn - NSLOT, sn).wait()
            gather(jn, sn).start()
        gather(j, s).wait()
        writeback(j, s).start()
    for j in range(NCHUNK - NSLOT, NCHUNK):
        writeback(j, j % NSLOT).wait()


@jax.jit
def kernel(x, tables):
    tab_flat = tables.reshape(NUM_FIELDS * VOCAB, EMBED_DIM)
    x_resh = x.reshape(NW, NCHUNK, CHUNK)
    offs = jnp.tile(jnp.arange(NUM_FIELDS, dtype=jnp.int32) * VOCAB,
                    RPW // NUM_FIELDS).reshape(NCHUNK, CHUNK)

    fn = pl.kernel(
        _body,
        out_type=jax.ShapeDtypeStruct((NW, RPW, EMBED_DIM), jnp.float32),
        mesh=plsc.VectorSubcoreMesh(core_axis_name="c", subcore_axis_name="s"),
        scratch_types=[
            pltpu.VMEM((NCHUNK, CHUNK), jnp.int32),
            pltpu.VMEM((NCHUNK, CHUNK), jnp.int32),
            [pltpu.VMEM((CHUNK, EMBED_DIM), jnp.float32) for _ in range(NSLOT)],
            [pltpu.SemaphoreType.DMA for _ in range(NSLOT)],
            [pltpu.SemaphoreType.DMA for _ in range(NSLOT)],
        ],
        compiler_params=pltpu.CompilerParams(use_tc_tiling_on_sc=False),
    )
    out = fn(x_resh, offs, tab_flat)
    return out.reshape(BATCH, 1, NUM_FIELDS * EMBED_DIM)


# speedup vs baseline: 2.2912x; 1.8900x over previous
import jax
import jax.numpy as jnp
from jax import lax
from jax.experimental import pallas as pl
from jax.experimental.pallas import tpu as pltpu
from jax.experimental.pallas import tpu_sc as plsc

NUM_FIELDS = 26
VOCAB = 100000
EMBED_DIM = 32
BATCH = 16384

_info = plsc.get_sparse_core_info()
NC, NS, L = _info.num_cores, _info.num_subcores, _info.num_lanes
NW = NC * NS
H0 = 49920
H1 = VOCAB - H0
SLAB = 4096
NSLAB = BATCH // SLAB
UNROLL = 8


def _body(xt_hbm, tt_hbm, out_hbm, tv0, tv1, xf, ob, sem_t, sem_x, sem_o):
    e = lax.axis_index("s") * NC + lax.axis_index("c")

    def t_copy(f, half):
        if half == 0:
            return pltpu.make_async_copy(
                tt_hbm.at[f, e, pl.ds(0, H0)], tv0, sem_t)
        return pltpu.make_async_copy(
            tt_hbm.at[f, e, pl.ds(H0, H1)], tv1, sem_t)

    def x_copy(f, s, par):
        return pltpu.make_async_copy(
            xt_hbm.at[f, pl.ds(s * SLAB, SLAB)], xf.at[par], sem_x)

    def o_copy(f, s):
        return pltpu.make_async_copy(
            ob.at[pl.ds(s * SLAB, SLAB)],
            out_hbm.at[f * EMBED_DIM + e, pl.ds(s * SLAB, SLAB)], sem_o)

    def compute(s, par, second):
        base = s * SLAB

        def step(i, _):
            for u in range(UNROLL):
                o = (i * UNROLL + u) * L
                sl = pl.ds(o, L)
                idx = xf[par, sl]
                if not second:
                    g = plsc.load_gather(tv0, [jnp.minimum(idx, H0 - 1)])
                    ob[pl.ds(base + o, L)] = g
                else:
                    g = plsc.load_gather(
                        tv1, [jnp.minimum(jnp.maximum(idx - H0, 0), H1 - 1)])
                    prev = ob[pl.ds(base + o, L)]
                    ob[pl.ds(base + o, L)] = jnp.where(idx >= H0, g, prev)
            return 0

        lax.fori_loop(0, SLAB // L // UNROLL, step, 0)

    t_copy(0, 0).start()
    x_copy(0, 0, 0).start()

    def field(f, _):
        t_copy(f, 0).wait()
        t_copy(f, 1).start()
        for s in range(NSLAB):
            par = s % 2
            x_copy(f, s, par).wait()
            x_copy(f, (s + 1) % NSLAB, (s + 1) % 2).start()

            @pl.when(f > 0)
            def _():
                o_copy(f - 1, s).wait()

            compute(s, par, second=False)
        t_copy(f, 1).wait()

        @pl.when(f + 1 < NUM_FIELDS)
        def _():
            t_copy(f + 1, 0).start()

        for s in range(NSLAB):
            par = s % 2
            x_copy(f, s, par).wait()
            if s + 1 < NSLAB:
                x_copy(f, s + 1, (s + 1) % 2).start()
            else:
                @pl.when(f + 1 < NUM_FIELDS)
                def _():
                    x_copy(f + 1, 0, 0).start()
            compute(s, par, second=True)
            o_copy(f, s).start()
        return 0

    lax.fori_loop(0, NUM_FIELDS, field, 0)
    for s in range(NSLAB):
        o_copy(NUM_FIELDS - 1, s).wait()


@jax.jit
def kernel(x, tables):
    xt = x.T
    tt = jnp.swapaxes(tables, 1, 2)
    fn = pl.kernel(
        _body,
        out_type=jax.ShapeDtypeStruct((NUM_FIELDS * EMBED_DIM, BATCH),
                                      jnp.float32),
        mesh=plsc.VectorSubcoreMesh(core_axis_name="c", subcore_axis_name="s"),
        scratch_types=[
            pltpu.VMEM((H0,), jnp.float32),
            pltpu.VMEM((H1,), jnp.float32),
            pltpu.VMEM((2, SLAB), jnp.int32),
            pltpu.VMEM((BATCH,), jnp.float32),
            pltpu.SemaphoreType.DMA,
            pltpu.SemaphoreType.DMA,
            pltpu.SemaphoreType.DMA,
        ],
        compiler_params=pltpu.CompilerParams(needs_layout_passes=False),
    )
    out_t = fn(xt, tt)
    return out_t.T.reshape(BATCH, 1, NUM_FIELDS * EMBED_DIM)
